# Initial kernel scaffold; baseline (speedup 1.0000x reference)
import jax
import jax.numpy as jnp
from jax.experimental import pallas as pl


def kernel(inputs, edge_index, h0, lamda, alpha, l, W):
    raise NotImplementedError("write your pallas kernel here")


# speedup vs baseline: 10.2954x; 10.2954x over previous
import functools

import jax
import jax.numpy as jnp
from jax import lax
from jax.experimental import pallas as pl
from jax.experimental.pallas import tpu as pltpu
from jax.experimental.pallas import tpu_sc as plsc

N = 10000
E = 320000
D = 128

NC = 2
NS = 16
NW = NC * NS

K = 128
EPW = ((E + NW * K - 1) // (NW * K)) * K
E_PAD = EPW * NW
CHUNKS = EPW // K

PAD_ROWS = 368
N_ACC = N + PAD_ROWS
ZR = N_ACC // NS
OUTR = 632
N_OUT = OUTR * NS

R_BLK = 2000
GRID = N // R_BLK

_MESH = plsc.VectorSubcoreMesh(core_axis_name="c", subcore_axis_name="s")


@functools.partial(
    pl.kernel,
    out_type=jax.ShapeDtypeStruct((NC, N_OUT, D), jnp.float32),
    mesh=_MESH,
    scratch_types=[
        pltpu.VMEM_SHARED((N_ACC, D), jnp.float32),
        pltpu.VMEM((K,), jnp.int32),
        pltpu.VMEM((K, D), jnp.float32),
    ],
)
def _sc_degree(dst_hbm, zeros_hbm, ones_hbm, deg_out, deg_acc, didx, ones):
    cid = lax.axis_index("c")
    sid = lax.axis_index("s")
    wid = sid * NC + cid

    pltpu.sync_copy(ones_hbm, ones)
    pltpu.sync_copy(zeros_hbm.at[pl.ds(sid * ZR, ZR)],
                    deg_acc.at[pl.ds(sid * ZR, ZR)])
    plsc.subcore_barrier()

    base0 = wid * EPW

    def chunk(k, carry):
        b = pl.multiple_of(base0 + k * K, 8)
        pltpu.sync_copy(dst_hbm.at[pl.ds(b, K)], didx)
        pltpu.sync_copy(ones, deg_acc.at[didx], add=True)
        return carry

    lax.fori_loop(0, CHUNKS, chunk, 0)
    plsc.subcore_barrier()
    r0 = sid * OUTR
    pltpu.sync_copy(deg_acc.at[pl.ds(r0, OUTR)],
                    deg_out.at[cid, pl.ds(r0, OUTR)])


@functools.partial(
    pl.kernel,
    out_type=jax.ShapeDtypeStruct((NC, N_OUT, D), jnp.float32),
    mesh=_MESH,
    scratch_types=[
        pltpu.VMEM_SHARED((N_ACC, D), jnp.float32),
        pltpu.VMEM((K,), jnp.int32),
        pltpu.VMEM((K,), jnp.int32),
        pltpu.VMEM((K, D), jnp.float32),
        pltpu.SemaphoreType.DMA,
    ],
)
def _sc_scatter(x_hbm, src_hbm, dst_hbm, zeros_hbm, part_out,
                acc, sidx, didx, rows, gsem):
    cid = lax.axis_index("c")
    sid = lax.axis_index("s")
    wid = sid * NC + cid

    pltpu.sync_copy(zeros_hbm.at[pl.ds(sid * ZR, ZR)],
                    acc.at[pl.ds(sid * ZR, ZR)])
    plsc.subcore_barrier()

    base0 = wid * EPW

    def chunk(k, carry):
        b = pl.multiple_of(base0 + k * K, 8)
        pltpu.sync_copy(src_hbm.at[pl.ds(b, K)], sidx)
        pltpu.sync_copy(dst_hbm.at[pl.ds(b, K)], didx)
        pltpu.async_copy(x_hbm.at[sidx], rows, gsem).wait()
        pltpu.sync_copy(rows, acc.at[didx], add=True)
        return carry

    lax.fori_loop(0, CHUNKS, chunk, 0)
    plsc.subcore_barrier()
    r0 = sid * OUTR
    pltpu.sync_copy(acc.at[pl.ds(r0, OUTR)],
                    part_out.at[cid, pl.ds(r0, OUTR)])


def _tc_scale_body(x_ref, deg_ref, out_ref):
    d = deg_ref[0, :, 0:1] + deg_ref[1, :, 0:1]
    dis = lax.rsqrt(jnp.maximum(d, 1.0))
    out_ref[...] = x_ref[...] * dis


def _tc_scale(x, deg_parts):
    return pl.pallas_call(
        _tc_scale_body,
        grid=(GRID,),
        in_specs=[
            pl.BlockSpec((R_BLK, D), lambda i: (i, 0)),
            pl.BlockSpec((NC, R_BLK, D), lambda i: (0, i, 0)),
        ],
        out_specs=pl.BlockSpec((R_BLK, D), lambda i: (i, 0)),
        out_shape=jax.ShapeDtypeStruct((N, D), jnp.float32),
    )(x, deg_parts)


def _tc_final_body(scal_ref, part_ref, deg_ref, h0_ref, w_ref, out_ref):
    theta = scal_ref[0, 0]
    alpha = scal_ref[0, 1]
    d = deg_ref[0, :, 0:1] + deg_ref[1, :, 0:1]
    dis = lax.rsqrt(jnp.maximum(d, 1.0))
    h_acc = part_ref[0] + part_ref[1]
    support = (1.0 - alpha) * (h_acc * dis) + alpha * h0_ref[...]
    mm = jnp.dot(support, w_ref[...], preferred_element_type=jnp.float32)
    out_ref[...] = theta * mm + (1.0 - theta) * support


def _tc_final(part, deg_parts, h0, W, scal):
    return pl.pallas_call(
        _tc_final_body,
        grid=(GRID,),
        in_specs=[
            pl.BlockSpec(memory_space=pltpu.SMEM),
            pl.BlockSpec((NC, R_BLK, D), lambda i: (0, i, 0)),
            pl.BlockSpec((NC, R_BLK, D), lambda i: (0, i, 0)),
            pl.BlockSpec((R_BLK, D), lambda i: (i, 0)),
            pl.BlockSpec((D, D), lambda i: (0, 0)),
        ],
        out_specs=pl.BlockSpec((R_BLK, D), lambda i: (i, 0)),
        out_shape=jax.ShapeDtypeStruct((N, D), jnp.float32),
    )(scal, part, deg_parts, h0, W)


def kernel(inputs, edge_index, h0, lamda, alpha, l, W):
    theta = jnp.log(lamda / l + 1)
    scal = jnp.reshape(
        jnp.stack([theta, alpha]).astype(jnp.float32), (1, 2))

    src = edge_index[0]
    dst = edge_index[1]
    npad = E_PAD - E
    pad_ar = jnp.arange(npad, dtype=jnp.int32)
    src_p = jnp.concatenate([src, pad_ar % N])
    dst_p = jnp.concatenate([dst, N + (pad_ar % PAD_ROWS)])

    zeros128 = jnp.zeros((N_ACC, D), jnp.float32)
    ones128 = jnp.ones((K, D), jnp.float32)

    deg_parts = _sc_degree(dst_p, zeros128, ones128)
    x_scaled = _tc_scale(inputs, deg_parts)
    part = _sc_scatter(x_scaled, src_p, dst_p, zeros128)
    return _tc_final(part, deg_parts, h0, W, scal)


# speedup vs baseline: 16.2422x; 1.5776x over previous
import functools

import jax
import jax.numpy as jnp
from jax import lax
from jax.experimental import pallas as pl
from jax.experimental.pallas import tpu as pltpu
from jax.experimental.pallas import tpu_sc as plsc

N = 10000
E = 320000
D = 128

NC = 2
NS = 16
NW = NC * NS

K = 128
EPW = ((E + NW * K - 1) // (NW * K)) * K
E_PAD = EPW * NW
CHUNKS = EPW // K

PAD_ROWS = 368
N_ACC = N + PAD_ROWS
ZR = N_ACC // NS
OUTR = 632
N_OUT = OUTR * NS

R_BLK = 2000
GRID = N // R_BLK

_MESH = plsc.VectorSubcoreMesh(core_axis_name="c", subcore_axis_name="s")


@functools.partial(
    pl.kernel,
    out_type=jax.ShapeDtypeStruct((NC, N_OUT, D), jnp.float32),
    mesh=_MESH,
    scratch_types=[
        pltpu.VMEM_SHARED((N_ACC, D), jnp.float32),
        pltpu.VMEM((K,), jnp.int32),
        pltpu.VMEM((K,), jnp.int32),
        pltpu.VMEM((K, D), jnp.float32),
        pltpu.SemaphoreType.DMA,
        pltpu.SemaphoreType.DMA,
    ],
)
def _sc_degree(dst_hbm, zeros_hbm, ones_hbm, deg_out, deg_acc,
               didx0, didx1, ones, isem0, isem1):
    cid = lax.axis_index("c")
    sid = lax.axis_index("s")
    wid = sid * NC + cid

    pltpu.sync_copy(ones_hbm, ones)
    pltpu.sync_copy(zeros_hbm.at[pl.ds(sid * ZR, ZR)],
                    deg_acc.at[pl.ds(sid * ZR, ZR)])
    plsc.subcore_barrier()

    base0 = wid * EPW
    dbufs = ((didx0, isem0), (didx1, isem1))

    def edge_slice(k):
        return dst_hbm.at[pl.ds(pl.multiple_of(base0 + k * K, 8), K)]

    pltpu.async_copy(edge_slice(0), didx0, isem0)
    pltpu.async_copy(edge_slice(1), didx1, isem1)

    def pair(g, carry):
        for b in range(2):
            k = 2 * g + b
            didx, isem = dbufs[b]
            pltpu.make_async_copy(edge_slice(k), didx, isem).wait()
            pltpu.sync_copy(ones, deg_acc.at[didx], add=True)

            @pl.when(k + 2 < CHUNKS)
            def _():
                pltpu.async_copy(edge_slice(k + 2), didx, isem)
        return carry

    lax.fori_loop(0, CHUNKS // 2, pair, 0)
    if CHUNKS % 2:
        k = CHUNKS - 1
        didx, isem = dbufs[k % 2]
        pltpu.make_async_copy(edge_slice(k), didx, isem).wait()
        pltpu.sync_copy(ones, deg_acc.at[didx], add=True)
    plsc.subcore_barrier()
    r0 = sid * OUTR
    pltpu.sync_copy(deg_acc.at[pl.ds(r0, OUTR)],
                    deg_out.at[cid, pl.ds(r0, OUTR)])


@functools.partial(
    pl.kernel,
    out_type=jax.ShapeDtypeStruct((NC, N_OUT, D), jnp.float32),
    mesh=_MESH,
    scratch_types=[
        pltpu.VMEM_SHARED((N_ACC, D), jnp.float32),
        pltpu.VMEM((K,), jnp.int32),
        pltpu.VMEM((K,), jnp.int32),
        pltpu.VMEM((K,), jnp.int32),
        pltpu.VMEM((K,), jnp.int32),
        pltpu.VMEM((K, D), jnp.float32),
        pltpu.VMEM((K, D), jnp.float32),
        pltpu.SemaphoreType.DMA,
        pltpu.SemaphoreType.DMA,
        pltpu.SemaphoreType.DMA,
        pltpu.SemaphoreType.DMA,
    ],
)
def _sc_scatter(x_hbm, src_hbm, dst_hbm, zeros_hbm, part_out,
                acc, sidx0, sidx1, didx0, didx1, rows0, rows1,
                isem0, isem1, gsem0, gsem1):
    cid = lax.axis_index("c")
    sid = lax.axis_index("s")
    wid = sid * NC + cid

    pltpu.sync_copy(zeros_hbm.at[pl.ds(sid * ZR, ZR)],
                    acc.at[pl.ds(sid * ZR, ZR)])
    plsc.subcore_barrier()

    base0 = wid * EPW
    bufs = ((sidx0, didx0, rows0, isem0, gsem0),
            (sidx1, didx1, rows1, isem1, gsem1))

    def src_slice(k):
        return src_hbm.at[pl.ds(pl.multiple_of(base0 + k * K, 8), K)]

    def dst_slice(k):
        return dst_hbm.at[pl.ds(pl.multiple_of(base0 + k * K, 8), K)]

    def issue_idx(k, b):
        sidx, didx, _, isem, _ = bufs[b]
        pltpu.async_copy(src_slice(k), sidx, isem)
        pltpu.async_copy(dst_slice(k), didx, isem)

    def wait_idx(k, b):
        sidx, didx, _, isem, _ = bufs[b]
        pltpu.make_async_copy(src_slice(k), sidx, isem).wait()
        pltpu.make_async_copy(dst_slice(k), didx, isem).wait()

    def issue_gather(b):
        sidx, _, rows, _, gsem = bufs[b]
        pltpu.async_copy(x_hbm.at[sidx], rows, gsem)

    def wait_gather(b):
        sidx, _, rows, _, gsem = bufs[b]
        pltpu.make_async_copy(x_hbm.at[sidx], rows, gsem).wait()

    def scatter(b):
        _, didx, rows, _, _ = bufs[b]
        pltpu.sync_copy(rows, acc.at[didx], add=True)

    issue_idx(0, 0)
    issue_idx(1, 1)
    wait_idx(0, 0)
    issue_gather(0)

    def pair(g, carry):
        for b in range(2):
            k = 2 * g + b
            nb = 1 - b

            @pl.when(k < CHUNKS)
            def _():
                wait_gather(b)

                @pl.when(k + 1 < CHUNKS)
                def _():
                    wait_idx(k + 1, nb)
                    issue_gather(nb)

                scatter(b)

                @pl.when(k + 2 < CHUNKS)
                def _():
                    issue_idx(k + 2, b)
        return carry

    lax.fori_loop(0, (CHUNKS + 1) // 2, pair, 0)
    plsc.subcore_barrier()
    r0 = sid * OUTR
    pltpu.sync_copy(acc.at[pl.ds(r0, OUTR)],
                    part_out.at[cid, pl.ds(r0, OUTR)])


def _tc_scale_body(x_ref, deg_ref, out_ref):
    d = deg_ref[0, :, 0:1] + deg_ref[1, :, 0:1]
    dis = lax.rsqrt(jnp.maximum(d, 1.0))
    out_ref[...] = x_ref[...] * dis


def _tc_scale(x, deg_parts):
    return pl.pallas_call(
        _tc_scale_body,
        grid=(GRID,),
        in_specs=[
            pl.BlockSpec((R_BLK, D), lambda i: (i, 0)),
            pl.BlockSpec((NC, R_BLK, D), lambda i: (0, i, 0)),
        ],
        out_specs=pl.BlockSpec((R_BLK, D), lambda i: (i, 0)),
        out_shape=jax.ShapeDtypeStruct((N, D), jnp.float32),
    )(x, deg_parts)


def _tc_final_body(scal_ref, part_ref, deg_ref, h0_ref, w_ref, out_ref):
    theta = scal_ref[0, 0]
    alpha = scal_ref[0, 1]
    d = deg_ref[0, :, 0:1] + deg_ref[1, :, 0:1]
    dis = lax.rsqrt(jnp.maximum(d, 1.0))
    h_acc = part_ref[0] + part_ref[1]
    support = (1.0 - alpha) * (h_acc * dis) + alpha * h0_ref[...]
    mm = jnp.dot(support, w_ref[...], preferred_element_type=jnp.float32)
    out_ref[...] = theta * mm + (1.0 - theta) * support


def _tc_final(part, deg_parts, h0, W, scal):
    return pl.pallas_call(
        _tc_final_body,
        grid=(GRID,),
        in_specs=[
            pl.BlockSpec(memory_space=pltpu.SMEM),
            pl.BlockSpec((NC, R_BLK, D), lambda i: (0, i, 0)),
            pl.BlockSpec((NC, R_BLK, D), lambda i: (0, i, 0)),
            pl.BlockSpec((R_BLK, D), lambda i: (i, 0)),
            pl.BlockSpec((D, D), lambda i: (0, 0)),
        ],
        out_specs=pl.BlockSpec((R_BLK, D), lambda i: (i, 0)),
        out_shape=jax.ShapeDtypeStruct((N, D), jnp.float32),
    )(scal, part, deg_parts, h0, W)


def kernel(inputs, edge_index, h0, lamda, alpha, l, W):
    theta = jnp.log(lamda / l + 1)
    scal = jnp.reshape(
        jnp.stack([theta, alpha]).astype(jnp.float32), (1, 2))

    src = edge_index[0]
    dst = edge_index[1]
    npad = E_PAD - E
    pad_ar = jnp.arange(npad, dtype=jnp.int32)
    src_p = jnp.concatenate([src, pad_ar % N])
    dst_p = jnp.concatenate([dst, N + (pad_ar % PAD_ROWS)])

    zeros128 = jnp.zeros((N_ACC, D), jnp.float32)
    ones128 = jnp.ones((K, D), jnp.float32)

    deg_parts = _sc_degree(dst_p, zeros128, ones128)
    x_scaled = _tc_scale(inputs, deg_parts)
    part = _sc_scatter(x_scaled, src_p, dst_p, zeros128)
    return _tc_final(part, deg_parts, h0, W, scal)


# speedup vs baseline: 16.9580x; 1.0441x over previous
import functools

import jax
import jax.numpy as jnp
from jax import lax
from jax.experimental import pallas as pl
from jax.experimental.pallas import tpu as pltpu
from jax.experimental.pallas import tpu_sc as plsc

N = 10000
E = 320000
D = 128

NC = 2
NS = 16
NW = NC * NS

K = 120
EPW = ((E + NW * K - 1) // (NW * K)) * K
E_PAD = EPW * NW
CHUNKS = EPW // K

PAD_ROWS = 112
N_ACC = N + PAD_ROWS
ZR = N_ACC // NS
OUTR = 632
OUTR_LAST = N - 15 * OUTR

R_BLK = 2000
GRID = N // R_BLK

_MESH = plsc.VectorSubcoreMesh(core_axis_name="c", subcore_axis_name="s")


def _writeback(sid, cid, acc, out_hbm):
    r0 = sid * OUTR

    @pl.when(sid < NS - 1)
    def _():
        pltpu.sync_copy(acc.at[pl.ds(r0, OUTR)],
                        out_hbm.at[cid, pl.ds(r0, OUTR)])

    @pl.when(sid == NS - 1)
    def _():
        r1 = (NS - 1) * OUTR
        pltpu.sync_copy(acc.at[pl.ds(r1, OUTR_LAST)],
                        out_hbm.at[cid, pl.ds(r1, OUTR_LAST)])


@functools.partial(
    pl.kernel,
    out_type=jax.ShapeDtypeStruct((NC, N, D), jnp.float32),
    mesh=_MESH,
    scratch_types=[
        pltpu.VMEM_SHARED((N_ACC, D), jnp.float32),
        pltpu.VMEM((K,), jnp.int32),
        pltpu.VMEM((K,), jnp.int32),
        pltpu.VMEM((K, D), jnp.float32),
        pltpu.SemaphoreType.DMA,
        pltpu.SemaphoreType.DMA,
    ],
)
def _sc_degree(dst_hbm, zeros_hbm, ones_hbm, deg_out, deg_acc,
               didx0, didx1, ones, isem0, isem1):
    cid = lax.axis_index("c")
    sid = lax.axis_index("s")
    wid = sid * NC + cid

    pltpu.sync_copy(ones_hbm, ones)
    pltpu.sync_copy(zeros_hbm.at[pl.ds(sid * ZR, ZR)],
                    deg_acc.at[pl.ds(sid * ZR, ZR)])
    plsc.subcore_barrier()

    base0 = wid * EPW
    dbufs = ((didx0, isem0), (didx1, isem1))

    def edge_slice(k):
        return dst_hbm.at[pl.ds(pl.multiple_of(base0 + k * K, 8), K)]

    pltpu.async_copy(edge_slice(0), didx0, isem0)
    pltpu.async_copy(edge_slice(1), didx1, isem1)

    def pair(g, carry):
        for b in range(2):
            k = 2 * g + b
            didx, isem = dbufs[b]
            pltpu.make_async_copy(edge_slice(k), didx, isem).wait()
            pltpu.sync_copy(ones, deg_acc.at[didx], add=True)

            @pl.when(k + 2 < CHUNKS)
            def _():
                pltpu.async_copy(edge_slice(k + 2), didx, isem)
        return carry

    lax.fori_loop(0, CHUNKS // 2, pair, 0)
    plsc.subcore_barrier()
    _writeback(sid, cid, deg_acc, deg_out)


@functools.partial(
    pl.kernel,
    out_type=jax.ShapeDtypeStruct((NC, N, D), jnp.float32),
    mesh=_MESH,
    scratch_types=(
        [pltpu.VMEM_SHARED((N_ACC, D), jnp.float32)]
        + [pltpu.VMEM((K,), jnp.int32)] * 6
        + [pltpu.VMEM((K, D), jnp.float32)] * 3
        + [pltpu.SemaphoreType.DMA] * 6
    ),
)
def _sc_scatter(x_hbm, src_hbm, dst_hbm, zeros_hbm, part_out, acc,
                sidx0, sidx1, sidx2, didx0, didx1, didx2,
                rows0, rows1, rows2,
                isem0, isem1, isem2, gsem0, gsem1, gsem2):
    cid = lax.axis_index("c")
    sid = lax.axis_index("s")
    wid = sid * NC + cid

    pltpu.sync_copy(zeros_hbm.at[pl.ds(sid * ZR, ZR)],
                    acc.at[pl.ds(sid * ZR, ZR)])
    plsc.subcore_barrier()

    base0 = wid * EPW
    bufs = ((sidx0, didx0, rows0, isem0, gsem0),
            (sidx1, didx1, rows1, isem1, gsem1),
            (sidx2, didx2, rows2, isem2, gsem2))
    NB = 3

    def src_slice(k):
        return src_hbm.at[pl.ds(pl.multiple_of(base0 + k * K, 8), K)]

    def dst_slice(k):
        return dst_hbm.at[pl.ds(pl.multiple_of(base0 + k * K, 8), K)]

    def issue_idx(k, b):
        sidx, didx, _, isem, _ = bufs[b]
        pltpu.async_copy(src_slice(k), sidx, isem)
        pltpu.async_copy(dst_slice(k), didx, isem)

    def wait_idx(k, b):
        sidx, didx, _, isem, _ = bufs[b]
        pltpu.make_async_copy(src_slice(k), sidx, isem).wait()
        pltpu.make_async_copy(dst_slice(k), didx, isem).wait()

    def issue_gather(b):
        sidx, _, rows, _, gsem = bufs[b]
        pltpu.async_copy(x_hbm.at[sidx], rows, gsem)

    def wait_gather(b):
        sidx, _, rows, _, gsem = bufs[b]
        pltpu.make_async_copy(x_hbm.at[sidx], rows, gsem).wait()

    def scatter(b):
        _, didx, rows, _, _ = bufs[b]
        pltpu.sync_copy(rows, acc.at[didx], add=True)

    for b in range(NB):
        issue_idx(b, b)
    for b in range(NB - 1):
        wait_idx(b, b)
        issue_gather(b)

    def triple(g, carry):
        for b in range(NB):
            k = NB * g + b

            @pl.when(k < CHUNKS)
            def _():
                wait_gather(b)
                nb = (b + NB - 1) % NB

                @pl.when(k + NB - 1 < CHUNKS)
                def _():
                    wait_idx(k + NB - 1, nb)
                    issue_gather(nb)

                scatter(b)

                @pl.when(k + NB < CHUNKS)
                def _():
                    issue_idx(k + NB, b)
        return carry

    lax.fori_loop(0, (CHUNKS + NB - 1) // NB, triple, 0)
    plsc.subcore_barrier()
    _writeback(sid, cid, acc, part_out)


def _tc_scale_body(x_ref, deg_ref, out_ref):
    d = deg_ref[0, :, 0:1] + deg_ref[1, :, 0:1]
    dis = lax.rsqrt(jnp.maximum(d, 1.0))
    out_ref[...] = x_ref[...] * dis


def _tc_scale(x, deg_parts):
    return pl.pallas_call(
        _tc_scale_body,
        grid=(GRID,),
        in_specs=[
            pl.BlockSpec((R_BLK, D), lambda i: (i, 0)),
            pl.BlockSpec((NC, R_BLK, D), lambda i: (0, i, 0)),
        ],
        out_specs=pl.BlockSpec((R_BLK, D), lambda i: (i, 0)),
        out_shape=jax.ShapeDtypeStruct((N, D), jnp.float32),
    )(x, deg_parts)


def _tc_final_body(scal_ref, part_ref, deg_ref, h0_ref, w_ref, out_ref):
    theta = scal_ref[0, 0]
    alpha = scal_ref[0, 1]
    d = deg_ref[0, :, 0:1] + deg_ref[1, :, 0:1]
    dis = lax.rsqrt(jnp.maximum(d, 1.0))
    h_acc = part_ref[0] + part_ref[1]
    support = (1.0 - alpha) * (h_acc * dis) + alpha * h0_ref[...]
    mm = jnp.dot(support, w_ref[...], preferred_element_type=jnp.float32)
    out_ref[...] = theta * mm + (1.0 - theta) * support


def _tc_final(part, deg_parts, h0, W, scal):
    return pl.pallas_call(
        _tc_final_body,
        grid=(GRID,),
        in_specs=[
            pl.BlockSpec(memory_space=pltpu.SMEM),
            pl.BlockSpec((NC, R_BLK, D), lambda i: (0, i, 0)),
            pl.BlockSpec((NC, R_BLK, D), lambda i: (0, i, 0)),
            pl.BlockSpec((R_BLK, D), lambda i: (i, 0)),
            pl.BlockSpec((D, D), lambda i: (0, 0)),
        ],
        out_specs=pl.BlockSpec((R_BLK, D), lambda i: (i, 0)),
        out_shape=jax.ShapeDtypeStruct((N, D), jnp.float32),
    )(scal, part, deg_parts, h0, W)


def kernel(inputs, edge_index, h0, lamda, alpha, l, W):
    theta = jnp.log(lamda / l + 1)
    scal = jnp.reshape(
        jnp.stack([theta, alpha]).astype(jnp.float32), (1, 2))

    src = edge_index[0]
    dst = edge_index[1]
    npad = E_PAD - E
    pad_ar = jnp.arange(npad, dtype=jnp.int32)
    src_p = jnp.concatenate([src, pad_ar % N])
    dst_p = jnp.concatenate([dst, N + (pad_ar % PAD_ROWS)])

    zeros128 = jnp.zeros((N_ACC, D), jnp.float32)
    ones128 = jnp.ones((K, D), jnp.float32)

    deg_parts = _sc_degree(dst_p, zeros128, ones128)
    x_scaled = _tc_scale(inputs, deg_parts)
    part = _sc_scatter(x_scaled, src_p, dst_p, zeros128)
    return _tc_final(part, deg_parts, h0, W, scal)
